# baseline (speedup 1.0000x reference)
import functools

import jax
import jax.numpy as jnp
from jax import lax
from jax.experimental import pallas as pl
from jax.experimental.pallas import tpu as pltpu
from jax.experimental.pallas import tpu_sc as plsc

_NC = 2
_NS = 16
_NW = _NC * _NS
_B = 4096
_S = 200
_E = 64
_C = 32
_ROWS_PER_W = _B // _NW
_IDX_PER_W = _ROWS_PER_W * _S
_CHUNK = 128
_NCHUNK = _IDX_PER_W // _CHUNK
_RING = 8


def _sc_pool_sum(idx3, table, dest, zeros):
    mesh = plsc.VectorSubcoreMesh(core_axis_name="c", subcore_axis_name="s")

    @functools.partial(
        pl.kernel,
        out_type=jax.ShapeDtypeStruct((_NW, _ROWS_PER_W, _E), jnp.float32),
        mesh=mesh,
        scratch_types=[
            pltpu.VMEM((_NCHUNK, _CHUNK), jnp.int32),
            pltpu.VMEM((_NCHUNK, _CHUNK), jnp.int32),
            pltpu.VMEM((_RING, _CHUNK, _E), jnp.float32),
            pltpu.VMEM_SHARED((_NS * _ROWS_PER_W, _E), jnp.float32),
        ] + [pltpu.SemaphoreType.DMA] * _RING,
        compiler_params=pltpu.CompilerParams(use_tc_tiling_on_sc=False),
    )
    def k(idx_hbm, tbl_hbm, dst_hbm, zero_hbm, out_hbm,
          idx_v, dest_v, ring_v, acc_sh, *sems):
        cid = lax.axis_index("c")
        sid = lax.axis_index("s")
        wid = cid * _NS + sid

        pltpu.sync_copy(idx_hbm.at[wid], idx_v)
        pltpu.sync_copy(dst_hbm.at[sid], dest_v)
        pltpu.sync_copy(zero_hbm, acc_sh.at[pl.ds(sid * _ROWS_PER_W, _ROWS_PER_W)])

        def fire(j, b):
            pltpu.async_copy(tbl_hbm.at[idx_v.at[j]], ring_v.at[b], sems[b])

        for b in range(_RING):
            fire(b, b)

        @pl.loop(0, _NCHUNK - _RING, step=_RING)
        def _(j0):
            for b in range(_RING):
                j = j0 + b
                pltpu.make_async_copy(
                    tbl_hbm.at[idx_v.at[j]], ring_v.at[b], sems[b]).wait()
                pltpu.sync_copy(ring_v.at[b], acc_sh.at[dest_v.at[j]], add=True)
                fire(j + _RING, b)

        for b in range(_RING):
            j = _NCHUNK - _RING + b
            pltpu.make_async_copy(
                tbl_hbm.at[idx_v.at[j]], ring_v.at[b], sems[b]).wait()
            pltpu.sync_copy(ring_v.at[b], acc_sh.at[dest_v.at[j]], add=True)

        pltpu.sync_copy(
            acc_sh.at[pl.ds(sid * _ROWS_PER_W, _ROWS_PER_W)], out_hbm.at[wid])

    return k(idx3, table, dest, zeros)


def _tc_head(pooled_sum, W, b2):
    blk = 512

    def body(p_ref, w_ref, b_ref, o_ref):
        x = p_ref[...] * (1.0 / _S)
        logits = jnp.dot(x, w_ref[...], preferred_element_type=jnp.float32)
        logits = logits + b_ref[...]
        m = jnp.max(logits, axis=-1, keepdims=True)
        e = jnp.exp(logits - m)
        o_ref[...] = e / jnp.sum(e, axis=-1, keepdims=True)

    return pl.pallas_call(
        body,
        grid=(_B // blk,),
        in_specs=[
            pl.BlockSpec((blk, _E), lambda i: (i, 0)),
            pl.BlockSpec((_E, _C), lambda i: (0, 0)),
            pl.BlockSpec((1, _C), lambda i: (0, 0)),
        ],
        out_specs=pl.BlockSpec((blk, _C), lambda i: (i, 0)),
        out_shape=jax.ShapeDtypeStruct((_B, _C), jnp.float32),
    )(pooled_sum, W, b2)


def kernel(indices, table, W, b):
    idx3 = indices.astype(jnp.int32).reshape(_NW, _NCHUNK, _CHUNK)
    base = (jnp.arange(_IDX_PER_W, dtype=jnp.int32) // _S).reshape(_NCHUNK, _CHUNK)
    dest = base[None, :, :] + (
        jnp.arange(_NS, dtype=jnp.int32) * _ROWS_PER_W)[:, None, None]
    zeros = jnp.zeros((_ROWS_PER_W, _E), jnp.float32)
    pooled_sum = _sc_pool_sum(idx3, table, dest, zeros).reshape(_B, _E)
    return _tc_head(pooled_sum, W, b.reshape(1, _C))
